# Initial kernel scaffold; baseline (speedup 1.0000x reference)
import jax
import jax.numpy as jnp
from jax.experimental import pallas as pl


def kernel(z, edge_index, edge_weight, W_msg, b_msg, W_u1, b_u1, W_u2, b_u2):
    raise NotImplementedError("write your pallas kernel here")


# speedup vs baseline: 2.4532x; 2.4532x over previous
import functools

import jax
import jax.numpy as jnp
from jax import lax
from jax.experimental import pallas as pl
from jax.experimental.pallas import tpu as pltpu
from jax.experimental.pallas import tpu_sc as plsc

N = 10000
E = 320000
D = 128

NW = 32
NPW = 320
NPAD = NW * NPW
CAP = 20480
CHUNK = 2000
NCHUNK = E // CHUNK
GRP = CHUNK // 16
BATCH = 128
NEG = -3.0e38
EMPTY_THRESH = -1.0e30


def _sc_segment_max(bm, dst, src, w, r):
  mesh = plsc.VectorSubcoreMesh(core_axis_name="c", subcore_axis_name="s")

  @functools.partial(
      pl.kernel,
      out_type=jax.ShapeDtypeStruct((NPAD * D,), jnp.float32),
      mesh=mesh,
      compiler_params=pltpu.CompilerParams(needs_layout_passes=False),
      scratch_types=[
          pltpu.VMEM((CAP,), jnp.int32),
          pltpu.VMEM((CAP,), jnp.int32),
          pltpu.VMEM((CAP,), jnp.float32),
          pltpu.VMEM((CHUNK,), jnp.int32),
          pltpu.VMEM((CHUNK,), jnp.int32),
          pltpu.VMEM((CHUNK,), jnp.float32),
          pltpu.VMEM((BATCH,), jnp.int32),
          pltpu.VMEM((D,), jnp.float32),
          pltpu.VMEM(((NPW + 1) * D,), jnp.float32),
          pltpu.VMEM((BATCH, D), jnp.float32),
          pltpu.SemaphoreType.DMA,
      ],
  )
  def k(bm_hbm, dst_hbm, src_hbm, w_hbm, r_hbm, out_hbm,
        seld, sels, selw, dchunk, schunk, wchunk, idxbuf, r_v, acc, rows,
        sem):
    wid = lax.axis_index("c") * 16 + lax.axis_index("s")
    base = wid * NPW

    pltpu.sync_copy(r_hbm, r_v)

    negv = jnp.full((16,), NEG, jnp.float32)
    zi = jnp.zeros((16,), jnp.int32)
    zf = jnp.zeros((16,), jnp.float32)
    padd = jnp.full((16,), NPW, jnp.int32)

    def init_acc(i, carry):
      acc[pl.ds(i * 16, 16)] = negv
      return carry

    lax.fori_loop(0, (NPW + 1) * D // 16, init_acc, 0)

    def init_sel(i, carry):
      s16 = pl.ds(i * 16, 16)
      seld[s16] = padd
      sels[s16] = zi
      selw[s16] = zf
      return carry

    lax.fori_loop(0, CAP // 16, init_sel, 0)

    def chunk_body(g, cnt):
      off = g * CHUNK
      pltpu.sync_copy(dst_hbm.at[pl.ds(off, CHUNK)], dchunk)
      pltpu.sync_copy(src_hbm.at[pl.ds(off, CHUNK)], schunk)
      pltpu.sync_copy(w_hbm.at[pl.ds(off, CHUNK)], wchunk)

      def grp_body(i, cnt):
        s16 = pl.ds(i * 16, 16)
        dl = dchunk[s16] - base
        m = dl.astype(jnp.uint32) < jnp.uint32(NPW)
        pc = plsc.all_reduce_population_count(m)[0]
        dst_slot = pl.ds(cnt, 16)
        plsc.store_compressed(seld.at[dst_slot], dl, mask=m)
        plsc.store_compressed(sels.at[dst_slot], schunk[s16], mask=m)
        plsc.store_compressed(selw.at[dst_slot], wchunk[s16], mask=m)
        return cnt + pc

      return lax.fori_loop(0, GRP, grp_body, cnt)

    cnt = lax.fori_loop(0, NCHUNK, chunk_body, jnp.int32(0))

    nb = (cnt + BATCH - 1) // BATCH

    def batch_body(b, carry):
      bo = b * BATCH
      for q in range(BATCH // 16):
        idxbuf[pl.ds(q * 16, 16)] = sels[pl.ds(bo + q * 16, 16)]
      pltpu.async_copy(bm_hbm.at[idxbuf], rows, sem).wait()

      def grp16_body(g, carry):
        dvec = seld[pl.ds(bo + g * 16, 16)]
        wvec = selw[pl.ds(bo + g * 16, 16)]
        for l in range(16):
          rowoff = dvec[l] * D
          wv = wvec[l]
          j = g * 16 + l
          for cc in range(D // 16):
            a = acc[pl.ds(rowoff + cc * 16, 16)]
            bv = rows[j, pl.ds(cc * 16, 16)]
            val = bv + wv * r_v[pl.ds(cc * 16, 16)]
            acc[pl.ds(rowoff + cc * 16, 16)] = jnp.maximum(a, val)
        return carry

      lax.fori_loop(0, BATCH // 16, grp16_body, 0)
      return carry

    lax.fori_loop(0, nb, batch_body, 0)

    pltpu.sync_copy(acc.at[pl.ds(0, NPW * D)],
                    out_hbm.at[pl.ds(base * D, NPW * D)])

  return k(bm, dst, src, w, r)


def _tc_pre_body(z_ref, w1_ref, w2_ref, bmsg_ref, a_ref, b_ref):
  zb = z_ref[...]
  a_ref[...] = (jnp.dot(zb, w1_ref[...], preferred_element_type=jnp.float32)
                + bmsg_ref[...])
  b_ref[...] = jnp.dot(zb, w2_ref[...], preferred_element_type=jnp.float32)


def _tc_post_body(z_ref, a_ref, mx_ref, w1a_ref, w1b_ref, b1_ref, w2_ref,
                  b2_ref, o_ref):
  mx = mx_ref[...]
  agg = jnp.where(mx > EMPTY_THRESH, a_ref[...] + mx, 0.0)
  zb = z_ref[...]
  hid = jnp.maximum(
      jnp.dot(zb, w1a_ref[...], preferred_element_type=jnp.float32)
      + jnp.dot(agg, w1b_ref[...], preferred_element_type=jnp.float32)
      + b1_ref[...], 0.0)
  o_ref[...] = (jnp.dot(hid, w2_ref[...], preferred_element_type=jnp.float32)
                + b2_ref[...])


_ROWS_BLK = 1000
_W_SPEC = pl.BlockSpec((D, D), lambda i: (0, 0))
_B_SPEC = pl.BlockSpec((1, D), lambda i: (0, 0))
_Z_SPEC = pl.BlockSpec((_ROWS_BLK, D), lambda i: (i, 0))


def kernel(z, edge_index, edge_weight, W_msg, b_msg, W_u1, b_u1, W_u2, b_u2):
  dst = edge_index[0]
  src = edge_index[1]
  w1 = W_msg[:D]
  w2 = W_msg[D:2 * D]
  r = W_msg[2 * D]

  a_mat, b_mat = pl.pallas_call(
      _tc_pre_body,
      grid=(N // _ROWS_BLK,),
      in_specs=[_Z_SPEC, _W_SPEC, _W_SPEC, _B_SPEC],
      out_specs=[_Z_SPEC, _Z_SPEC],
      out_shape=[jax.ShapeDtypeStruct((N, D), jnp.float32)] * 2,
  )(z, w1, w2, b_msg.reshape(1, D))

  mx = _sc_segment_max(b_mat, dst, src, edge_weight, r).reshape(NPAD, D)[:N]

  h = pl.pallas_call(
      _tc_post_body,
      grid=(N // _ROWS_BLK,),
      in_specs=[_Z_SPEC, _Z_SPEC, _Z_SPEC, _W_SPEC, _W_SPEC, _B_SPEC,
                _W_SPEC, _B_SPEC],
      out_specs=_Z_SPEC,
      out_shape=jax.ShapeDtypeStruct((N, D), jnp.float32),
  )(z, a_mat, mx, W_u1[:D], W_u1[D:], b_u1.reshape(1, D), W_u2,
    b_u2.reshape(1, D))
  return h
